# Initial kernel scaffold; baseline (speedup 1.0000x reference)
import jax
import jax.numpy as jnp
from jax.experimental import pallas as pl


def kernel(h_states, seq_start_end, end_pos, W1, b1, gamma, beta):
    raise NotImplementedError("write your pallas kernel here")


# speedup vs baseline: 46.1369x; 46.1369x over previous
import functools

import jax
import jax.numpy as jnp
from jax import lax
from jax.experimental import pallas as pl
from jax.experimental.pallas import tpu as pltpu

H_DIM = 64
GRID = 8
NS = 2.0
PED = 64
TG = GRID * GRID
POOL_IN = TG * H_DIM


def _scene_kernel(h_ref, pos_ref, ox_ref, oy_ref, w1_ref, b1_ref,
                  y_ref, stats_ref):
    s = pl.program_id(0)

    pos = pos_ref[...]
    ax = pos[:, 0:1]
    ay = pos[:, 1:2]
    ox = ox_ref[0]
    oy = oy_ref[0]

    tlx = ax - NS / 2.0
    tly = ay + NS / 2.0
    brx = ax + NS / 2.0
    bry = ay - NS / 2.0

    cell_x = jnp.floor((ox - tlx) / NS * GRID)
    cell_y = jnp.floor((tly - oy) / NS * GRID)
    x_bound = (ox >= brx) | (ox <= tlx)
    y_bound = (oy >= tly) | (oy <= bry)
    eye = (lax.broadcasted_iota(jnp.int32, (PED, PED), 0)
           == lax.broadcasted_iota(jnp.int32, (PED, PED), 1))
    within = x_bound | y_bound | eye
    gp = jnp.where(within, -1.0, cell_x + cell_y * GRID).astype(jnp.int32)

    h = h_ref[...]
    z = jax.lax.dot(h, w1_ref[...], preferred_element_type=jnp.float32)
    zf = jnp.concatenate([z[:, c * H_DIM:(c + 1) * H_DIM] for c in range(TG)],
                         axis=0)

    gpt = jnp.tile(gp, (1, TG))
    kc = lax.broadcasted_iota(jnp.int32, (PED, TG * PED), 1) // PED
    m2 = (gpt == kc).astype(jnp.float32)

    y = jax.lax.dot(m2, zf,
                    preferred_element_type=jnp.float32) + b1_ref[0:1, :]
    y_ref[...] = y

    @pl.when(s == 0)
    def _():
        stats_ref[...] = jnp.zeros_like(stats_ref)

    stats_ref[0:1, :] += jnp.sum(y, axis=0, keepdims=True)
    stats_ref[1:2, :] += jnp.sum(y * y, axis=0, keepdims=True)


def _norm_kernel(y_ref, stats_ref, gamma_ref, beta_ref, out_ref, *, n_rows):
    mu = stats_ref[0:1, :] * (1.0 / n_rows)
    ex2 = stats_ref[1:2, :] * (1.0 / n_rows)
    var = ex2 - mu * mu
    inv = 1.0 / jnp.sqrt(var + 1e-5)
    y = y_ref[...]
    out = (y - mu) * (inv * gamma_ref[0:1, :]) + beta_ref[0:1, :]
    out_ref[...] = jnp.maximum(out, 0.0)


def kernel(h_states, seq_start_end, end_pos, W1, b1, gamma, beta):
    n = h_states.shape[0]
    num_seqs = seq_start_end.shape[0]
    assert n == num_seqs * PED

    ox = end_pos[:, 0].reshape(num_seqs, 1, PED)
    oy = end_pos[:, 1].reshape(num_seqs, 1, PED)
    w1p = W1.reshape(TG, H_DIM, H_DIM).transpose(1, 0, 2).reshape(H_DIM, POOL_IN)
    b1r = jnp.broadcast_to(b1.reshape(1, H_DIM), (8, H_DIM))
    gammar = jnp.broadcast_to(gamma.reshape(1, H_DIM), (8, H_DIM))
    betar = jnp.broadcast_to(beta.reshape(1, H_DIM), (8, H_DIM))

    y, stats = pl.pallas_call(
        _scene_kernel,
        grid=(num_seqs,),
        in_specs=[
            pl.BlockSpec((PED, H_DIM), lambda s: (s, 0)),
            pl.BlockSpec((PED, 2), lambda s: (s, 0)),
            pl.BlockSpec((1, 1, PED), lambda s: (s, 0, 0)),
            pl.BlockSpec((1, 1, PED), lambda s: (s, 0, 0)),
            pl.BlockSpec((H_DIM, POOL_IN), lambda s: (0, 0)),
            pl.BlockSpec((8, H_DIM), lambda s: (0, 0)),
        ],
        out_specs=[
            pl.BlockSpec((PED, H_DIM), lambda s: (s, 0)),
            pl.BlockSpec((8, H_DIM), lambda s: (0, 0)),
        ],
        out_shape=[
            jax.ShapeDtypeStruct((n, H_DIM), jnp.float32),
            jax.ShapeDtypeStruct((8, H_DIM), jnp.float32),
        ],
    )(h_states, end_pos, ox, oy, w1p, b1r)

    rows_blk = 2048
    out = pl.pallas_call(
        functools.partial(_norm_kernel, n_rows=n),
        grid=(n // rows_blk,),
        in_specs=[
            pl.BlockSpec((rows_blk, H_DIM), lambda r: (r, 0)),
            pl.BlockSpec((8, H_DIM), lambda r: (0, 0)),
            pl.BlockSpec((8, H_DIM), lambda r: (0, 0)),
            pl.BlockSpec((8, H_DIM), lambda r: (0, 0)),
        ],
        out_specs=pl.BlockSpec((rows_blk, H_DIM), lambda r: (r, 0)),
        out_shape=jax.ShapeDtypeStruct((n, H_DIM), jnp.float32),
    )(y, stats, gammar, betar)
    return out


# speedup vs baseline: 63.8321x; 1.3835x over previous
import functools

import jax
import jax.numpy as jnp
from jax import lax
from jax.experimental import pallas as pl
from jax.experimental.pallas import tpu as pltpu

H_DIM = 64
GRID = 8
NS = 2.0
PED = 64
TG = GRID * GRID
POOL_IN = TG * H_DIM
SB = 4


def _scene_kernel(h_ref, pos_ref, ox_ref, oy_ref, w1_ref, b1_ref,
                  y_ref, stats_ref):
    step = pl.program_id(0)

    z = jax.lax.dot(h_ref[...], w1_ref[...],
                    preferred_element_type=jnp.float32)
    zb = z.astype(jnp.bfloat16)

    kc = lax.broadcasted_iota(jnp.int32, (PED, TG * PED), 1) // PED

    for s in range(SB):
        pos = pos_ref[s * PED:(s + 1) * PED, :]
        ax = pos[:, 0:1]
        ay = pos[:, 1:2]
        ox = ox_ref[s]
        oy = oy_ref[s]

        tlx = ax - NS / 2.0
        tly = ay + NS / 2.0
        brx = ax + NS / 2.0
        bry = ay - NS / 2.0

        cell_x = jnp.floor((ox - tlx) / NS * GRID)
        cell_y = jnp.floor((tly - oy) / NS * GRID)
        x_bound = (ox >= brx) | (ox <= tlx)
        y_bound = (oy >= tly) | (oy <= bry)
        eye = (lax.broadcasted_iota(jnp.int32, (PED, PED), 0)
               == lax.broadcasted_iota(jnp.int32, (PED, PED), 1))
        within = x_bound | y_bound | eye
        gp = jnp.where(within, -1.0, cell_x + cell_y * GRID).astype(jnp.int32)
        spill = jnp.concatenate(
            [jnp.full((1, PED), -128, jnp.int32), gp[:PED - 1, :] - 64],
            axis=0)

        zs = zb[s * PED:(s + 1) * PED, :]
        zf = jnp.concatenate(
            [zs[:, c * H_DIM:(c + 1) * H_DIM] for c in range(TG)],
            axis=0)

        gpt = jnp.tile(gp, (1, TG))
        spt = jnp.tile(spill, (1, TG))
        m2 = ((gpt == kc).astype(jnp.bfloat16)
              + (spt == kc).astype(jnp.bfloat16))

        y = jax.lax.dot(m2, zf,
                        preferred_element_type=jnp.float32) + b1_ref[0:1, :]
        y_ref[s * PED:(s + 1) * PED, :] = y

        @pl.when((step == 0) & (s == 0))
        def _():
            stats_ref[...] = jnp.zeros_like(stats_ref)

        stats_ref[0:1, :] += jnp.sum(y, axis=0, keepdims=True)
        stats_ref[1:2, :] += jnp.sum(y * y, axis=0, keepdims=True)


def _norm_kernel(y_ref, stats_ref, gamma_ref, beta_ref, out_ref, *, n_rows):
    mu = stats_ref[0:1, :] * (1.0 / n_rows)
    ex2 = stats_ref[1:2, :] * (1.0 / n_rows)
    var = ex2 - mu * mu
    inv = 1.0 / jnp.sqrt(var + 1e-5)
    y = y_ref[...]
    out = (y - mu) * (inv * gamma_ref[0:1, :]) + beta_ref[0:1, :]
    out_ref[...] = jnp.maximum(out, 0.0)


def kernel(h_states, seq_start_end, end_pos, W1, b1, gamma, beta):
    n = h_states.shape[0]
    num_seqs = seq_start_end.shape[0]
    assert n == num_seqs * PED

    ox = end_pos[:, 0].reshape(num_seqs, 1, PED)
    oy = end_pos[:, 1].reshape(num_seqs, 1, PED)
    w1p = W1.reshape(TG, H_DIM, H_DIM).transpose(1, 0, 2).reshape(
        H_DIM, POOL_IN).astype(jnp.bfloat16)
    hb = h_states.astype(jnp.bfloat16)
    b1r = jnp.broadcast_to(b1.reshape(1, H_DIM), (8, H_DIM))
    gammar = jnp.broadcast_to(gamma.reshape(1, H_DIM), (8, H_DIM))
    betar = jnp.broadcast_to(beta.reshape(1, H_DIM), (8, H_DIM))

    y, stats = pl.pallas_call(
        _scene_kernel,
        grid=(num_seqs // SB,),
        in_specs=[
            pl.BlockSpec((SB * PED, H_DIM), lambda t: (t, 0)),
            pl.BlockSpec((SB * PED, 2), lambda t: (t, 0)),
            pl.BlockSpec((SB, 1, PED), lambda t: (t, 0, 0)),
            pl.BlockSpec((SB, 1, PED), lambda t: (t, 0, 0)),
            pl.BlockSpec((H_DIM, POOL_IN), lambda t: (0, 0)),
            pl.BlockSpec((8, H_DIM), lambda t: (0, 0)),
        ],
        out_specs=[
            pl.BlockSpec((SB * PED, H_DIM), lambda t: (t, 0)),
            pl.BlockSpec((8, H_DIM), lambda t: (0, 0)),
        ],
        out_shape=[
            jax.ShapeDtypeStruct((n, H_DIM), jnp.float32),
            jax.ShapeDtypeStruct((8, H_DIM), jnp.float32),
        ],
    )(hb, end_pos, ox, oy, w1p, b1r)

    rows_blk = 2048
    out = pl.pallas_call(
        functools.partial(_norm_kernel, n_rows=n),
        grid=(n // rows_blk,),
        in_specs=[
            pl.BlockSpec((rows_blk, H_DIM), lambda r: (r, 0)),
            pl.BlockSpec((8, H_DIM), lambda r: (0, 0)),
            pl.BlockSpec((8, H_DIM), lambda r: (0, 0)),
            pl.BlockSpec((8, H_DIM), lambda r: (0, 0)),
        ],
        out_specs=pl.BlockSpec((rows_blk, H_DIM), lambda r: (r, 0)),
        out_shape=jax.ShapeDtypeStruct((n, H_DIM), jnp.float32),
    )(y, stats, gammar, betar)
    return out


# speedup vs baseline: 68.4481x; 1.0723x over previous
import functools

import jax
import jax.numpy as jnp
from jax import lax
from jax.experimental import pallas as pl
from jax.experimental.pallas import tpu as pltpu

H_DIM = 64
GRID = 8
NS = 2.0
PED = 64
TG = GRID * GRID
POOL_IN = TG * H_DIM
SB = 4
SBP = SB * PED
SPILL_K = 1024


def _scene_kernel(h_ref, pos_ref, ox_ref, oy_ref, w1_ref, kc_ref,
                  y_ref, stats_ref):
    step = pl.program_id(0)

    zb = jax.lax.dot(h_ref[...], w1_ref[...],
                     preferred_element_type=jnp.float32
                     ).astype(jnp.bfloat16)

    pos = pos_ref[...]
    ax = pos[:, 0:1]
    ay = pos[:, 1:2]
    ox = jnp.broadcast_to(ox_ref[...], (SB, PED, PED)).reshape(SBP, PED)
    oy = jnp.broadcast_to(oy_ref[...], (SB, PED, PED)).reshape(SBP, PED)

    tlx = ax - NS / 2.0
    tly = ay + NS / 2.0
    brx = ax + NS / 2.0
    bry = ay - NS / 2.0

    cell_x = jnp.floor((ox - tlx) / NS * GRID)
    cell_y = jnp.floor((tly - oy) / NS * GRID)
    x_bound = (ox >= brx) | (ox <= tlx)
    y_bound = (oy >= tly) | (oy <= bry)
    eye = ((lax.broadcasted_iota(jnp.int32, (SBP, PED), 0) % PED)
           == lax.broadcasted_iota(jnp.int32, (SBP, PED), 1))
    within = x_bound | y_bound | eye
    gp = jnp.where(within, -1.0, cell_x + cell_y * GRID).astype(jnp.int32)

    first_row = (lax.broadcasted_iota(jnp.int32, (SBP, PED), 0) % PED) == 0
    shifted = jnp.concatenate(
        [jnp.full((1, PED), -128, jnp.int32), gp[:SBP - 1, :] - 64], axis=0)
    spill = jnp.where(first_row, -128, shifted)

    kc = jnp.broadcast_to(kc_ref[0:1, :], (SBP, TG * PED))
    gpt = jnp.tile(gp, (1, TG))
    spt = jnp.tile(spill, (1, SPILL_K // PED))
    m_main = (gpt == kc).astype(jnp.bfloat16)
    m_low = (m_main[:, :SPILL_K]
             + (spt == kc[:, :SPILL_K]).astype(jnp.bfloat16))

    for s in range(SB):
        zs = zb[s * PED:(s + 1) * PED, :]
        zf = jnp.concatenate(
            [zs[:, c * H_DIM:(c + 1) * H_DIM] for c in range(TG)],
            axis=0)

        y = (jax.lax.dot(m_low[s * PED:(s + 1) * PED, :], zf[:SPILL_K, :],
                         preferred_element_type=jnp.float32)
             + jax.lax.dot(m_main[s * PED:(s + 1) * PED, SPILL_K:],
                           zf[SPILL_K:, :],
                           preferred_element_type=jnp.float32))
        y_ref[s * PED:(s + 1) * PED, :] = y

        @pl.when((step == 0) & (s == 0))
        def _():
            stats_ref[...] = jnp.zeros_like(stats_ref)

        stats_ref[0:1, :] += jnp.sum(y, axis=0, keepdims=True)
        stats_ref[1:2, :] += jnp.sum(y * y, axis=0, keepdims=True)


def _norm_kernel(y_ref, stats_ref, gamma_ref, beta_ref, out_ref, *, n_rows):
    mu = stats_ref[0:1, :] * (1.0 / n_rows)
    ex2 = stats_ref[1:2, :] * (1.0 / n_rows)
    var = ex2 - mu * mu
    inv = 1.0 / jnp.sqrt(var + 1e-5)
    y = y_ref[...]
    out = (y - mu) * (inv * gamma_ref[0:1, :]) + beta_ref[0:1, :]
    out_ref[...] = jnp.maximum(out, 0.0)


def kernel(h_states, seq_start_end, end_pos, W1, b1, gamma, beta):
    n = h_states.shape[0]
    num_seqs = seq_start_end.shape[0]
    assert n == num_seqs * PED

    ox = end_pos[:, 0].reshape(num_seqs, 1, PED)
    oy = end_pos[:, 1].reshape(num_seqs, 1, PED)
    w1p = W1.reshape(TG, H_DIM, H_DIM).transpose(1, 0, 2).reshape(
        H_DIM, POOL_IN).astype(jnp.bfloat16)
    hb = h_states.astype(jnp.bfloat16)
    kc8 = jnp.broadcast_to(
        (jnp.arange(TG * PED, dtype=jnp.int32) // PED).reshape(1, TG * PED),
        (8, TG * PED))
    gammar = jnp.broadcast_to(gamma.reshape(1, H_DIM), (8, H_DIM))
    betar = jnp.broadcast_to(beta.reshape(1, H_DIM), (8, H_DIM))

    y, stats = pl.pallas_call(
        _scene_kernel,
        grid=(num_seqs // SB,),
        in_specs=[
            pl.BlockSpec((SBP, H_DIM), lambda t: (t, 0)),
            pl.BlockSpec((SBP, 2), lambda t: (t, 0)),
            pl.BlockSpec((SB, 1, PED), lambda t: (t, 0, 0)),
            pl.BlockSpec((SB, 1, PED), lambda t: (t, 0, 0)),
            pl.BlockSpec((H_DIM, POOL_IN), lambda t: (0, 0)),
            pl.BlockSpec((8, TG * PED), lambda t: (0, 0)),
        ],
        out_specs=[
            pl.BlockSpec((SBP, H_DIM), lambda t: (t, 0)),
            pl.BlockSpec((8, H_DIM), lambda t: (0, 0)),
        ],
        out_shape=[
            jax.ShapeDtypeStruct((n, H_DIM), jnp.float32),
            jax.ShapeDtypeStruct((8, H_DIM), jnp.float32),
        ],
    )(hb, end_pos, ox, oy, w1p, kc8)

    rows_blk = 2048
    out = pl.pallas_call(
        functools.partial(_norm_kernel, n_rows=n),
        grid=(n // rows_blk,),
        in_specs=[
            pl.BlockSpec((rows_blk, H_DIM), lambda r: (r, 0)),
            pl.BlockSpec((8, H_DIM), lambda r: (0, 0)),
            pl.BlockSpec((8, H_DIM), lambda r: (0, 0)),
            pl.BlockSpec((8, H_DIM), lambda r: (0, 0)),
        ],
        out_specs=pl.BlockSpec((rows_blk, H_DIM), lambda r: (r, 0)),
        out_shape=jax.ShapeDtypeStruct((n, H_DIM), jnp.float32),
    )(y, stats, gammar, betar)
    return out
